# baseline (speedup 1.0000x reference)
import functools

import jax
import jax.numpy as jnp
import numpy as np
from jax import lax
from jax.experimental import pallas as pl
from jax.experimental.pallas import tpu as pltpu
from jax.experimental.pallas import tpu_sc as plsc

_N = 10000
_E = 320000
_D = 128
_BN_SCALE = float(1.0 / np.sqrt(1.0 + 1e-5))

_NC = 2
_NS = 16
_NW = _NC * _NS
_CH = 128
_CHA = 64
_NCH = 80
_NCHA = 160
_EPAD = _CH * _NCH * _NW
_RPT = 640
_NP = _NS * _RPT

_mesh = plsc.VectorSubcoreMesh(core_axis_name="c", subcore_axis_name="s")

_STAGE = ((0, 128), (128, 128), (256, 128), (384, 128), (512, 128))


def _fill_f32(ref, val16):
    def _body(i, carry):
        r = i // (_D // 16)
        k = i % (_D // 16)
        ref[r, pl.ds(k * 16, 16)] = val16
        return carry

    lax.fori_loop(0, _CH * _D // 16, _body, 0)


@functools.partial(
    pl.kernel,
    out_type=jax.ShapeDtypeStruct((_NC, _NP, _D), jnp.float32),
    mesh=_mesh,
    scratch_types=[
        pltpu.VMEM((_NCH, _CH), jnp.int32),
        pltpu.VMEM((_CH, _D), jnp.float32),
        pltpu.VMEM((_CH, _D), jnp.float32),
        pltpu.VMEM_SHARED((_NP, _D), jnp.float32),
    ],
)
def _sc_deg(dst_hbm, out_hbm, dst_v, ones_v, zst_v, acc):
    c = lax.axis_index("c")
    s = lax.axis_index("s")
    wid = s * _NC + c
    base = s * _RPT

    _fill_f32(ones_v, jnp.ones((16,), jnp.float32))
    _fill_f32(zst_v, jnp.zeros((16,), jnp.float32))

    for off, sz in _STAGE:
        pltpu.sync_copy(zst_v.at[pl.ds(0, sz)], acc.at[pl.ds(base + off, sz)])

    pltpu.sync_copy(dst_hbm.at[wid], dst_v)
    plsc.subcore_barrier()

    def _chunk(j, carry):
        pltpu.sync_copy(ones_v, acc.at[dst_v.at[j]], add=True)
        return carry

    lax.fori_loop(0, _NCH, _chunk, 0)
    plsc.subcore_barrier()

    for off, sz in _STAGE:
        pltpu.sync_copy(acc.at[pl.ds(base + off, sz)], zst_v.at[pl.ds(0, sz)])
        pltpu.sync_copy(zst_v.at[pl.ds(0, sz)], out_hbm.at[c, pl.ds(base + off, sz)])


_C0N = 320
_NLD = 4
_HT = _C0N // _NLD


@functools.partial(
    pl.kernel,
    out_type=jax.ShapeDtypeStruct((_NP, _D), jnp.float32),
    mesh=_mesh,
    scratch_types=[
        pltpu.VMEM((_HT, _CHA), jnp.int32),
        pltpu.VMEM((_HT, _CHA), jnp.int32),
        pltpu.VMEM((_CHA, _D), jnp.float32),
        pltpu.VMEM((_CHA, _D), jnp.float32),
        pltpu.VMEM_SHARED((_NP, _D), jnp.float32),
        pltpu.SemaphoreType.DMA,
        pltpu.SemaphoreType.DMA,
    ],
)
def _sc_agg(z_hbm, src_hbm, dst_hbm, out_hbm, src_v, dst_v, buf_a, buf_b, acc,
            sem_a, sem_b):
    c = lax.axis_index("c")
    s = lax.axis_index("s")
    base = s * _RPT

    @pl.when(c == 0)
    def _core0():
        def _zero64(ref):
            def _body(i, carry):
                r = i // (_D // 16)
                k = i % (_D // 16)
                ref[r, pl.ds(k * 16, 16)] = jnp.zeros((16,), jnp.float32)
                return carry
            lax.fori_loop(0, _CHA * _D // 16, _body, 0)

        _zero64(buf_a)
        for k in range(_RPT // _CHA):
            pltpu.sync_copy(buf_a, acc.at[pl.ds(base + k * _CHA, _CHA)])
        plsc.subcore_barrier()

        def _wait(buf, sem):
            pltpu.make_async_copy(z_hbm.at[pl.ds(0, _CHA)], buf, sem).wait()

        for h in range(_NLD):
            pltpu.sync_copy(src_hbm.at[s, pl.ds(h * _HT, _HT)], src_v)
            pltpu.sync_copy(dst_hbm.at[s, pl.ds(h * _HT, _HT)], dst_v)

            pltpu.async_copy(z_hbm.at[src_v.at[0]], buf_a, sem_a)
            pltpu.async_copy(z_hbm.at[src_v.at[1]], buf_b, sem_b)

            def _pair(j2, carry):
                j = j2 * 2
                _wait(buf_a, sem_a)
                pltpu.sync_copy(buf_a, acc.at[dst_v.at[j]], add=True)
                pltpu.async_copy(z_hbm.at[src_v.at[j + 2]], buf_a, sem_a)
                _wait(buf_b, sem_b)
                pltpu.sync_copy(buf_b, acc.at[dst_v.at[j + 1]], add=True)
                pltpu.async_copy(z_hbm.at[src_v.at[j + 3]], buf_b, sem_b)
                return carry

            lax.fori_loop(0, _HT // 2 - 1, _pair, 0)
            _wait(buf_a, sem_a)
            pltpu.sync_copy(buf_a, acc.at[dst_v.at[_HT - 2]], add=True)
            _wait(buf_b, sem_b)
            pltpu.sync_copy(buf_b, acc.at[dst_v.at[_HT - 1]], add=True)

        plsc.subcore_barrier()

        for k in range(_RPT // _CHA):
            pltpu.sync_copy(acc.at[pl.ds(base + k * _CHA, _CHA)], buf_a)
            pltpu.sync_copy(buf_a, out_hbm.at[pl.ds(base + k * _CHA, _CHA)])


def _dot(a, b):
    return jnp.dot(a, b, preferred_element_type=jnp.float32,
                   precision=lax.Precision.HIGHEST)


def _tc_first_body(x_ref, w_ref, degp_ref, z_ref, dinv_ref):
    deg = degp_ref[0, 0:_N, 0:1] + degp_ref[1, 0:_N, 0:1] + 1.0
    dinv = lax.rsqrt(deg)
    z_ref[...] = _dot(x_ref[...], w_ref[...]) * dinv
    dinv_ref[...] = dinv


_tc_first = pl.pallas_call(
    _tc_first_body,
    out_shape=(jax.ShapeDtypeStruct((_N, _D), jnp.float32),
               jax.ShapeDtypeStruct((_N, 1), jnp.float32)),
)


def _tc_mid_body(p_ref, z_ref, dinv_ref, b_ref, w_ref, zn_ref):
    dinv = dinv_ref[...]
    t = (p_ref[0:_N] + z_ref[...]) * dinv + b_ref[...]
    t = t * _BN_SCALE
    h = jnp.where(t > 0, t, 0.01 * t)
    zn_ref[...] = _dot(h, w_ref[...]) * dinv


_tc_mid = pl.pallas_call(
    _tc_mid_body,
    out_shape=jax.ShapeDtypeStruct((_N, _D), jnp.float32),
)


def _tc_last_body(p_ref, z_ref, dinv_ref, b_ref, out_ref):
    out_ref[...] = (p_ref[0:_N] + z_ref[...]) * dinv_ref[...] + b_ref[...]


_tc_last = pl.pallas_call(
    _tc_last_body,
    out_shape=jax.ShapeDtypeStruct((_N, _D), jnp.float32),
)


def kernel(x, edge_index, W1, b1, W2, b2, W3, b3):
    src = edge_index[0].astype(jnp.int32)
    dst = edge_index[1].astype(jnp.int32)
    pad = _EPAD - _E
    srcf = jnp.concatenate([src, jnp.zeros((pad,), jnp.int32)])
    dstf = jnp.concatenate([dst, jnp.full((pad,), _N, jnp.int32)])
    srcp = srcf.reshape(_NS, _C0N, _CHA)
    dstp = dstf.reshape(_NS, _C0N, _CHA)

    degp = _sc_deg(dstf.reshape(_NW, _NCH, _CH))
    z1, dinv = _tc_first(x, W1, degp)
    p1 = _sc_agg(z1, srcp, dstp)
    z2 = _tc_mid(p1, z1, dinv, b1.reshape(1, _D), W2)
    p2 = _sc_agg(z2, srcp, dstp)
    z3 = _tc_mid(p2, z2, dinv, b2.reshape(1, _D), W3)
    p3 = _sc_agg(z3, srcp, dstp)
    return _tc_last(p3, z3, dinv, b3.reshape(1, _D))


# speedup vs baseline: 1.3525x; 1.3525x over previous
import functools

import jax
import jax.numpy as jnp
import numpy as np
from jax import lax
from jax.experimental import pallas as pl
from jax.experimental.pallas import tpu as pltpu
from jax.experimental.pallas import tpu_sc as plsc

_N = 10000
_E = 320000
_D = 128
_BN_SCALE = float(1.0 / np.sqrt(1.0 + 1e-5))

_NC = 2
_NS = 16
_NW = _NC * _NS
_CH = 128
_CHA = 64
_NCH = 80
_NCHA = 160
_EPAD = _CH * _NCH * _NW
_RPT = 640
_NP = _NS * _RPT

_mesh = plsc.VectorSubcoreMesh(core_axis_name="c", subcore_axis_name="s")

_STAGE = ((0, 128), (128, 128), (256, 128), (384, 128), (512, 128))


def _fill_f32(ref, val16):
    def _body(i, carry):
        r = i // (_D // 16)
        k = i % (_D // 16)
        ref[r, pl.ds(k * 16, 16)] = val16
        return carry

    lax.fori_loop(0, _CH * _D // 16, _body, 0)


@functools.partial(
    pl.kernel,
    out_type=jax.ShapeDtypeStruct((_NC, _NP, _D), jnp.float32),
    mesh=_mesh,
    scratch_types=[
        pltpu.VMEM((_NCH, _CH), jnp.int32),
        pltpu.VMEM((_CH, _D), jnp.float32),
        pltpu.VMEM((_CH, _D), jnp.float32),
        pltpu.VMEM_SHARED((_NP, _D), jnp.float32),
    ],
)
def _sc_deg(dst_hbm, out_hbm, dst_v, ones_v, zst_v, acc):
    c = lax.axis_index("c")
    s = lax.axis_index("s")
    wid = s * _NC + c
    base = s * _RPT

    _fill_f32(ones_v, jnp.ones((16,), jnp.float32))
    _fill_f32(zst_v, jnp.zeros((16,), jnp.float32))

    for off, sz in _STAGE:
        pltpu.sync_copy(zst_v.at[pl.ds(0, sz)], acc.at[pl.ds(base + off, sz)])

    pltpu.sync_copy(dst_hbm.at[wid], dst_v)
    plsc.subcore_barrier()

    def _chunk(j, carry):
        pltpu.sync_copy(ones_v, acc.at[dst_v.at[j]], add=True)
        return carry

    lax.fori_loop(0, _NCH, _chunk, 0)
    plsc.subcore_barrier()

    for off, sz in _STAGE:
        pltpu.sync_copy(acc.at[pl.ds(base + off, sz)], zst_v.at[pl.ds(0, sz)])
        pltpu.sync_copy(zst_v.at[pl.ds(0, sz)], out_hbm.at[c, pl.ds(base + off, sz)])


_C0N = 240
_C1N = 80
_CAP = 240
_HT = _CAP // 2


@functools.partial(
    pl.kernel,
    out_type=jax.ShapeDtypeStruct((_NC, _NP, _D), jnp.float32),
    mesh=_mesh,
    scratch_types=[
        pltpu.VMEM((_HT, _CHA), jnp.int32),
        pltpu.VMEM((_HT, _CHA), jnp.int32),
        pltpu.VMEM((_CHA, _D), jnp.float32),
        pltpu.VMEM((_CHA, _D), jnp.float32),
        pltpu.VMEM_SHARED((_NP, _D), jnp.float32),
        pltpu.SemaphoreType.DMA,
        pltpu.SemaphoreType.DMA,
    ],
)
def _sc_agg(z_hbm, src_hbm, dst_hbm, out_hbm, src_v, dst_v, buf_a, buf_b, acc,
            sem_a, sem_b):
    c = lax.axis_index("c")
    s = lax.axis_index("s")
    wid = s * _NC + c
    base = s * _RPT
    nh = jnp.where(c == 0, _C0N // 2, _C1N // 2)

    def _zero64(ref):
        def _body(i, carry):
            r = i // (_D // 16)
            k = i % (_D // 16)
            ref[r, pl.ds(k * 16, 16)] = jnp.zeros((16,), jnp.float32)
            return carry
        lax.fori_loop(0, _CHA * _D // 16, _body, 0)

    _zero64(buf_a)
    for k in range(_RPT // _CHA):
        pltpu.sync_copy(buf_a, acc.at[pl.ds(base + k * _CHA, _CHA)])
    plsc.subcore_barrier()

    def _wait(buf, sem):
        pltpu.make_async_copy(z_hbm.at[pl.ds(0, _CHA)], buf, sem).wait()

    for h in range(2):
        pltpu.sync_copy(src_hbm.at[wid, pl.ds(h * nh, _HT)], src_v)
        pltpu.sync_copy(dst_hbm.at[wid, pl.ds(h * nh, _HT)], dst_v)

        pltpu.async_copy(z_hbm.at[src_v.at[0]], buf_a, sem_a)
        pltpu.async_copy(z_hbm.at[src_v.at[1]], buf_b, sem_b)

        def _pair(j2, carry):
            j = j2 * 2
            _wait(buf_a, sem_a)
            pltpu.sync_copy(buf_a, acc.at[dst_v.at[j]], add=True)
            pltpu.async_copy(z_hbm.at[src_v.at[j + 2]], buf_a, sem_a)
            _wait(buf_b, sem_b)
            pltpu.sync_copy(buf_b, acc.at[dst_v.at[j + 1]], add=True)
            pltpu.async_copy(z_hbm.at[src_v.at[j + 3]], buf_b, sem_b)
            return carry

        lax.fori_loop(0, nh // 2 - 1, _pair, 0)
        _wait(buf_a, sem_a)
        pltpu.sync_copy(buf_a, acc.at[dst_v.at[nh - 2]], add=True)
        _wait(buf_b, sem_b)
        pltpu.sync_copy(buf_b, acc.at[dst_v.at[nh - 1]], add=True)

    plsc.subcore_barrier()

    for k in range(_RPT // _CHA):
        pltpu.sync_copy(acc.at[pl.ds(base + k * _CHA, _CHA)], buf_a)
        pltpu.sync_copy(buf_a, out_hbm.at[c, pl.ds(base + k * _CHA, _CHA)])


def _dot(a, b):
    return jnp.dot(a, b, preferred_element_type=jnp.float32,
                   precision=lax.Precision.HIGHEST)


def _tc_first_body(x_ref, w_ref, degp_ref, z_ref, dinv_ref):
    deg = degp_ref[0, 0:_N, 0:1] + degp_ref[1, 0:_N, 0:1] + 1.0
    dinv = lax.rsqrt(deg)
    z_ref[...] = _dot(x_ref[...], w_ref[...]) * dinv
    dinv_ref[...] = dinv


_tc_first = pl.pallas_call(
    _tc_first_body,
    out_shape=(jax.ShapeDtypeStruct((_N, _D), jnp.float32),
               jax.ShapeDtypeStruct((_N, 1), jnp.float32)),
)


def _tc_mid_body(p_ref, z_ref, dinv_ref, b_ref, w_ref, zn_ref):
    dinv = dinv_ref[...]
    t = (p_ref[0, 0:_N] + p_ref[1, 0:_N] + z_ref[...]) * dinv + b_ref[...]
    t = t * _BN_SCALE
    h = jnp.where(t > 0, t, 0.01 * t)
    zn_ref[...] = _dot(h, w_ref[...]) * dinv


_tc_mid = pl.pallas_call(
    _tc_mid_body,
    out_shape=jax.ShapeDtypeStruct((_N, _D), jnp.float32),
)


def _tc_last_body(p_ref, z_ref, dinv_ref, b_ref, out_ref):
    out_ref[...] = (p_ref[0, 0:_N] + p_ref[1, 0:_N] + z_ref[...]) * dinv_ref[...] + b_ref[...]


_tc_last = pl.pallas_call(
    _tc_last_body,
    out_shape=jax.ShapeDtypeStruct((_N, _D), jnp.float32),
)


_E0 = _NS * _C0N * _CHA
_E1 = _NS * _C1N * _CHA


def _agg_table(flat, fill):
    c0 = flat[:_E0].reshape(_NS, _C0N, _CHA)
    c1 = flat[_E0:].reshape(_NS, _C1N, _CHA)
    c1 = jnp.pad(c1, ((0, 0), (0, _CAP - _C1N), (0, 0)), constant_values=fill)
    return jnp.stack([c0, c1], axis=1).reshape(_NW, _CAP, _CHA)


def kernel(x, edge_index, W1, b1, W2, b2, W3, b3):
    src = edge_index[0].astype(jnp.int32)
    dst = edge_index[1].astype(jnp.int32)
    pad = _EPAD - _E
    srcf = jnp.concatenate([src, jnp.zeros((pad,), jnp.int32)])
    dstf = jnp.concatenate([dst, jnp.full((pad,), _N, jnp.int32)])
    srcp = _agg_table(srcf, 0)
    dstp = _agg_table(dstf, _N)

    degp = _sc_deg(dstf.reshape(_NW, _NCH, _CH))
    z1, dinv = _tc_first(x, W1, degp)
    p1 = _sc_agg(z1, srcp, dstp)
    z2 = _tc_mid(p1, z1, dinv, b1.reshape(1, _D), W2)
    p2 = _sc_agg(z2, srcp, dstp)
    z3 = _tc_mid(p2, z2, dinv, b2.reshape(1, _D), W3)
    p3 = _sc_agg(z3, srcp, dstp)
    return _tc_last(p3, z3, dinv, b3.reshape(1, _D))


# speedup vs baseline: 1.4047x; 1.0386x over previous
import functools

import jax
import jax.numpy as jnp
import numpy as np
from jax import lax
from jax.experimental import pallas as pl
from jax.experimental.pallas import tpu as pltpu
from jax.experimental.pallas import tpu_sc as plsc

_N = 10000
_E = 320000
_D = 128
_BN_SCALE = float(1.0 / np.sqrt(1.0 + 1e-5))

_NC = 2
_NS = 16
_NW = _NC * _NS
_CH = 128
_CHA = 64
_NCH = 80
_NCHA = 160
_EPAD = _CH * _NCH * _NW
_RPT = 640
_NP = _NS * _RPT

_mesh = plsc.VectorSubcoreMesh(core_axis_name="c", subcore_axis_name="s")

_STAGE = ((0, 128), (128, 128), (256, 128), (384, 128), (512, 128))


def _fill_f32(ref, val16):
    def _body(i, carry):
        r = i // (_D // 16)
        k = i % (_D // 16)
        ref[r, pl.ds(k * 16, 16)] = val16
        return carry

    lax.fori_loop(0, _CH * _D // 16, _body, 0)


@functools.partial(
    pl.kernel,
    out_type=jax.ShapeDtypeStruct((_NC, _NP, _D), jnp.float32),
    mesh=_mesh,
    scratch_types=[
        pltpu.VMEM((_NCH, _CH), jnp.int32),
        pltpu.VMEM((_CH, _D), jnp.float32),
        pltpu.VMEM((_CH, _D), jnp.float32),
        pltpu.VMEM_SHARED((_NP, _D), jnp.float32),
    ],
)
def _sc_deg(dst_hbm, out_hbm, dst_v, ones_v, zst_v, acc):
    c = lax.axis_index("c")
    s = lax.axis_index("s")
    wid = s * _NC + c
    base = s * _RPT

    _fill_f32(ones_v, jnp.ones((16,), jnp.float32))
    _fill_f32(zst_v, jnp.zeros((16,), jnp.float32))

    for off, sz in _STAGE:
        pltpu.sync_copy(zst_v.at[pl.ds(0, sz)], acc.at[pl.ds(base + off, sz)])

    pltpu.sync_copy(dst_hbm.at[wid], dst_v)
    plsc.subcore_barrier()

    def _chunk(j, carry):
        pltpu.sync_copy(ones_v, acc.at[dst_v.at[j]], add=True)
        return carry

    lax.fori_loop(0, _NCH, _chunk, 0)
    plsc.subcore_barrier()

    for off, sz in _STAGE:
        pltpu.sync_copy(acc.at[pl.ds(base + off, sz)], zst_v.at[pl.ds(0, sz)])
        pltpu.sync_copy(zst_v.at[pl.ds(0, sz)], out_hbm.at[c, pl.ds(base + off, sz)])


_C0N = 288
_C1N = 32
_CAP = 288
_NLD = 4
_HT = _CAP // _NLD


@functools.partial(
    pl.kernel,
    out_type=jax.ShapeDtypeStruct((_NC, _NP, _D), jnp.float32),
    mesh=_mesh,
    scratch_types=[
        pltpu.VMEM((_HT, _CHA), jnp.int32),
        pltpu.VMEM((_HT, _CHA), jnp.int32),
        pltpu.VMEM((_CHA, _D), jnp.float32),
        pltpu.VMEM((_CHA, _D), jnp.float32),
        pltpu.VMEM_SHARED((_NP, _D), jnp.float32),
        pltpu.SemaphoreType.DMA,
        pltpu.SemaphoreType.DMA,
    ],
)
def _sc_agg(z_hbm, src_hbm, dst_hbm, out_hbm, src_v, dst_v, buf_a, buf_b, acc,
            sem_a, sem_b):
    c = lax.axis_index("c")
    s = lax.axis_index("s")
    wid = s * _NC + c
    base = s * _RPT
    nh = jnp.where(c == 0, _C0N // _NLD, _C1N // _NLD)

    def _zero64(ref):
        def _body(i, carry):
            r = i // (_D // 16)
            k = i % (_D // 16)
            ref[r, pl.ds(k * 16, 16)] = jnp.zeros((16,), jnp.float32)
            return carry
        lax.fori_loop(0, _CHA * _D // 16, _body, 0)

    _zero64(buf_a)
    for k in range(_RPT // _CHA):
        pltpu.sync_copy(buf_a, acc.at[pl.ds(base + k * _CHA, _CHA)])
    plsc.subcore_barrier()

    def _wait(buf, sem):
        pltpu.make_async_copy(z_hbm.at[pl.ds(0, _CHA)], buf, sem).wait()

    for h in range(_NLD):
        pltpu.sync_copy(src_hbm.at[wid, pl.ds(h * nh, _HT)], src_v)
        pltpu.sync_copy(dst_hbm.at[wid, pl.ds(h * nh, _HT)], dst_v)

        pltpu.async_copy(z_hbm.at[src_v.at[0]], buf_a, sem_a)
        pltpu.async_copy(z_hbm.at[src_v.at[1]], buf_b, sem_b)

        def _pair(j2, carry):
            j = j2 * 2
            _wait(buf_a, sem_a)
            pltpu.sync_copy(buf_a, acc.at[dst_v.at[j]], add=True)
            pltpu.async_copy(z_hbm.at[src_v.at[j + 2]], buf_a, sem_a)
            _wait(buf_b, sem_b)
            pltpu.sync_copy(buf_b, acc.at[dst_v.at[j + 1]], add=True)
            pltpu.async_copy(z_hbm.at[src_v.at[j + 3]], buf_b, sem_b)
            return carry

        lax.fori_loop(0, nh // 2 - 1, _pair, 0)
        _wait(buf_a, sem_a)
        pltpu.sync_copy(buf_a, acc.at[dst_v.at[nh - 2]], add=True)
        _wait(buf_b, sem_b)
        pltpu.sync_copy(buf_b, acc.at[dst_v.at[nh - 1]], add=True)

    plsc.subcore_barrier()

    for k in range(_RPT // _CHA):
        pltpu.sync_copy(acc.at[pl.ds(base + k * _CHA, _CHA)], buf_a)
        pltpu.sync_copy(buf_a, out_hbm.at[c, pl.ds(base + k * _CHA, _CHA)])


def _dot(a, b):
    return jnp.dot(a, b, preferred_element_type=jnp.float32,
                   precision=lax.Precision.HIGHEST)


def _tc_first_body(x_ref, w_ref, degp_ref, z_ref, dinv_ref):
    deg = degp_ref[0, 0:_N, 0:1] + degp_ref[1, 0:_N, 0:1] + 1.0
    dinv = lax.rsqrt(deg)
    z_ref[...] = _dot(x_ref[...], w_ref[...]) * dinv
    dinv_ref[...] = dinv


_tc_first = pl.pallas_call(
    _tc_first_body,
    out_shape=(jax.ShapeDtypeStruct((_N, _D), jnp.float32),
               jax.ShapeDtypeStruct((_N, 1), jnp.float32)),
)


def _tc_mid_body(p_ref, z_ref, dinv_ref, b_ref, w_ref, zn_ref):
    dinv = dinv_ref[...]
    t = (p_ref[0, 0:_N] + p_ref[1, 0:_N] + z_ref[...]) * dinv + b_ref[...]
    t = t * _BN_SCALE
    h = jnp.where(t > 0, t, 0.01 * t)
    zn_ref[...] = _dot(h, w_ref[...]) * dinv


_tc_mid = pl.pallas_call(
    _tc_mid_body,
    out_shape=jax.ShapeDtypeStruct((_N, _D), jnp.float32),
)


def _tc_last_body(p_ref, z_ref, dinv_ref, b_ref, out_ref):
    out_ref[...] = (p_ref[0, 0:_N] + p_ref[1, 0:_N] + z_ref[...]) * dinv_ref[...] + b_ref[...]


_tc_last = pl.pallas_call(
    _tc_last_body,
    out_shape=jax.ShapeDtypeStruct((_N, _D), jnp.float32),
)


_E0 = _NS * _C0N * _CHA
_E1 = _NS * _C1N * _CHA


def _agg_table(flat, fill):
    c0 = flat[:_E0].reshape(_NS, _C0N, _CHA)
    c1 = flat[_E0:].reshape(_NS, _C1N, _CHA)
    c1 = jnp.pad(c1, ((0, 0), (0, _CAP - _C1N), (0, 0)), constant_values=fill)
    return jnp.stack([c0, c1], axis=1).reshape(_NW, _CAP, _CHA)


def kernel(x, edge_index, W1, b1, W2, b2, W3, b3):
    src = edge_index[0].astype(jnp.int32)
    dst = edge_index[1].astype(jnp.int32)
    pad = _EPAD - _E
    srcf = jnp.concatenate([src, jnp.zeros((pad,), jnp.int32)])
    dstf = jnp.concatenate([dst, jnp.full((pad,), _N, jnp.int32)])
    srcp = _agg_table(srcf, 0)
    dstp = _agg_table(dstf, _N)

    degp = _sc_deg(dstf.reshape(_NW, _NCH, _CH))
    z1, dinv = _tc_first(x, W1, degp)
    p1 = _sc_agg(z1, srcp, dstp)
    z2 = _tc_mid(p1, z1, dinv, b1.reshape(1, _D), W2)
    p2 = _sc_agg(z2, srcp, dstp)
    z3 = _tc_mid(p2, z2, dinv, b2.reshape(1, _D), W3)
    p3 = _sc_agg(z3, srcp, dstp)
    return _tc_last(p3, z3, dinv, b3.reshape(1, _D))
